# initial kernel scaffold (unmeasured)
import jax
import jax.numpy as jnp
from jax import lax
from jax.experimental import pallas as pl
from jax.experimental.pallas import tpu as pltpu

N_DEV = 16
HEADS_PER = 8
SQ = 256
SKV = 4096
DH = 128
DM = 1024
DMODEL = 1024
CHUNK = SQ // N_DEV
SCALE = 0.08838834764831843


def kernel(x, Wq, K_ext, V_ext, Wo):
    def body(x_ref, wq_ref, k_hbm, v_hbm, wo_ref, out_ref,
             k_slab, v_slab, kv_sems, ctx_ref, partial_ref, rs_buf, ag_buf,
             ssem, rsem1, rsem2):
        me = lax.axis_index("i")

        h0 = me * HEADS_PER
        ck = pltpu.make_async_copy(
            k_hbm.at[0, :, pl.ds(h0, HEADS_PER), :], k_slab, kv_sems.at[0])
        cv = pltpu.make_async_copy(
            v_hbm.at[0, :, pl.ds(h0, HEADS_PER), :], v_slab, kv_sems.at[1])
        ck.start()
        cv.start()

        xb = x_ref[0].astype(jnp.bfloat16)
        wqb = wq_ref[...].astype(jnp.bfloat16)
        q_all = jnp.dot(xb, wqb, preferred_element_type=jnp.float32)
        q_all = q_all.astype(jnp.bfloat16).reshape(SQ, HEADS_PER, DH)

        qb = lax.broadcasted_iota(jnp.int32, (SQ, SKV), 0) // 64
        kb = (lax.broadcasted_iota(jnp.int32, (SQ, SKV), 1) // 64) % 4
        mask = qb == kb

        ck.wait()
        cv.wait()

        for h in range(HEADS_PER):
            kh = k_slab[:, h, :].astype(jnp.bfloat16)
            vh = v_slab[:, h, :].astype(jnp.bfloat16)
            qh = q_all[:, h, :]
            s = lax.dot_general(qh, kh, (((1,), (1,)), ((), ())),
                                preferred_element_type=jnp.float32) * SCALE
            s = jnp.where(mask, s, -1e9)
            m = jnp.max(s, axis=-1, keepdims=True)
            w = jnp.exp(s - m)
            denom = jnp.sum(w, axis=-1, keepdims=True)
            ctx_h = jnp.dot(w.astype(jnp.bfloat16), vh,
                            preferred_element_type=jnp.float32) / denom
            ctx_ref[:, h * DH:(h + 1) * DH] = ctx_h.astype(jnp.bfloat16)

        wob = wo_ref[...].astype(jnp.bfloat16)
        partial_ref[...] = jnp.dot(ctx_ref[...], wob,
                                   preferred_element_type=jnp.float32)

        rs_buf[pl.ds(me, 1)] = partial_ref[pl.ds(me * CHUNK, CHUNK), :].reshape(
            1, CHUNK, DMODEL)

        sends1 = []
        for o in range(1, N_DEV):
            peer = lax.rem(me + o, N_DEV)
            r = pltpu.make_async_remote_copy(
                src_ref=partial_ref.at[pl.ds(peer * CHUNK, CHUNK), :],
                dst_ref=rs_buf.at[me],
                send_sem=ssem.at[o - 1],
                recv_sem=rsem1.at[me],
                device_id=(peer,),
                device_id_type=pl.DeviceIdType.MESH,
            )
            r.start()
            sends1.append(r)

        for o in range(1, N_DEV):
            s_id = lax.rem(me + o, N_DEV)
            recv = pltpu.make_async_remote_copy(
                src_ref=partial_ref.at[pl.ds(0, CHUNK), :],
                dst_ref=rs_buf.at[s_id],
                send_sem=ssem.at[o - 1],
                recv_sem=rsem1.at[s_id],
                device_id=(me,),
                device_id_type=pl.DeviceIdType.MESH,
            )
            recv.wait_recv()

        acc = jnp.sum(rs_buf[...], axis=0)

        for r in sends1:
            r.wait_send()

        ag_buf[...] = acc
        out_ref[0, pl.ds(me * CHUNK, CHUNK), :] = acc

        sends2 = []
        for o in range(1, N_DEV):
            peer = lax.rem(me + o, N_DEV)
            r = pltpu.make_async_remote_copy(
                src_ref=ag_buf,
                dst_ref=out_ref.at[0, pl.ds(me * CHUNK, CHUNK), :],
                send_sem=ssem.at[o - 1],
                recv_sem=rsem2.at[me],
                device_id=(peer,),
                device_id_type=pl.DeviceIdType.MESH,
            )
            r.start()
            sends2.append(r)

        for o in range(1, N_DEV):
            s_id = lax.rem(me + o, N_DEV)
            recv = pltpu.make_async_remote_copy(
                src_ref=ag_buf,
                dst_ref=out_ref.at[0, pl.ds(s_id * CHUNK, CHUNK), :],
                send_sem=ssem.at[o - 1],
                recv_sem=rsem2.at[s_id],
                device_id=(me,),
                device_id_type=pl.DeviceIdType.MESH,
            )
            recv.wait_recv()

        for r in sends2:
            r.wait_send()

    return pl.pallas_call(
        body,
        out_shape=jax.ShapeDtypeStruct((1, SQ, DMODEL), jnp.float32),
        in_specs=[
            pl.BlockSpec(memory_space=pltpu.VMEM),
            pl.BlockSpec(memory_space=pltpu.VMEM),
            pl.BlockSpec(memory_space=pl.ANY),
            pl.BlockSpec(memory_space=pl.ANY),
            pl.BlockSpec(memory_space=pltpu.VMEM),
        ],
        out_specs=pl.BlockSpec(memory_space=pltpu.VMEM),
        scratch_shapes=[
            pltpu.VMEM((SKV, HEADS_PER, DH), jnp.float32),
            pltpu.VMEM((SKV, HEADS_PER, DH), jnp.float32),
            pltpu.SemaphoreType.DMA((2,)),
            pltpu.VMEM((SQ, DM), jnp.bfloat16),
            pltpu.VMEM((SQ, DMODEL), jnp.float32),
            pltpu.VMEM((N_DEV, CHUNK, DMODEL), jnp.float32),
            pltpu.VMEM((CHUNK, DMODEL), jnp.float32),
            pltpu.SemaphoreType.DMA((N_DEV,)),
            pltpu.SemaphoreType.DMA((N_DEV,)),
            pltpu.SemaphoreType.DMA((N_DEV,)),
        ],
    )(x, Wq, K_ext, V_ext, Wo)


# baseline (device time: 106212 ns/iter reference)
import jax
import jax.numpy as jnp
from jax import lax
from jax.experimental import pallas as pl
from jax.experimental.pallas import tpu as pltpu

N_DEV = 16
HEADS_PER = 8
SQ = 256
SKV = 4096
DH = 128
DM = 1024
DMODEL = 1024
CHUNK = SQ // N_DEV
SCALE = 0.08838834764831843


def kernel(x, Wq, K_ext, V_ext, Wo):
    def body(x_ref, wq_ref, k_hbm, v_hbm, wo_ref, out_ref,
             k_slab, v_slab, kv_sems, ctx_ref, partial_ref, rs_buf, ag_buf,
             ssem, rsem1, rsem2):
        me = lax.axis_index("i")

        h0 = me * HEADS_PER
        ck = pltpu.make_async_copy(
            k_hbm.at[0, :, pl.ds(h0, HEADS_PER), :], k_slab, kv_sems.at[0])
        cv = pltpu.make_async_copy(
            v_hbm.at[0, :, pl.ds(h0, HEADS_PER), :], v_slab, kv_sems.at[1])
        ck.start()
        cv.start()

        xb = x_ref[0].astype(jnp.bfloat16)
        wqb = wq_ref[...].astype(jnp.bfloat16)
        q_all = jnp.dot(xb, wqb, preferred_element_type=jnp.float32)
        q_all = q_all.astype(jnp.bfloat16).reshape(SQ, HEADS_PER, DH)

        qb = lax.broadcasted_iota(jnp.int32, (SQ, SKV), 0) // 64
        kb = (lax.broadcasted_iota(jnp.int32, (SQ, SKV), 1) // 64) % 4
        mask = qb == kb

        ck.wait()
        cv.wait()

        for h in range(HEADS_PER):
            kh = k_slab[:, h, :].astype(jnp.bfloat16)
            vh = v_slab[:, h, :].astype(jnp.bfloat16)
            qh = q_all[:, h, :]
            s = lax.dot_general(qh, kh, (((1,), (1,)), ((), ())),
                                preferred_element_type=jnp.float32) * SCALE
            s = jnp.where(mask, s, -1e9)
            m = jnp.max(s, axis=-1, keepdims=True)
            w = jnp.exp(s - m)
            denom = jnp.sum(w, axis=-1, keepdims=True)
            ctx_h = jnp.dot(w.astype(jnp.bfloat16), vh,
                            preferred_element_type=jnp.float32) / denom
            ctx_ref[:, h * DH:(h + 1) * DH] = ctx_h.astype(jnp.bfloat16)

        wob = wo_ref[...].astype(jnp.bfloat16)
        partial_ref[...] = jnp.dot(ctx_ref[...], wob,
                                   preferred_element_type=jnp.float32)

        rs_buf[pl.ds(me, 1)] = partial_ref[pl.ds(me * CHUNK, CHUNK), :].reshape(
            1, CHUNK, DMODEL)

        sends1 = []
        for o in range(1, N_DEV):
            peer = lax.rem(me + o, N_DEV)
            r = pltpu.make_async_remote_copy(
                src_ref=partial_ref.at[pl.ds(peer * CHUNK, CHUNK), :],
                dst_ref=rs_buf.at[me],
                send_sem=ssem.at[o - 1],
                recv_sem=rsem1.at[me],
                device_id=(peer,),
                device_id_type=pl.DeviceIdType.MESH,
            )
            r.start()
            sends1.append(r)

        for o in range(1, N_DEV):
            s_id = lax.rem(me + o, N_DEV)
            recv = pltpu.make_async_remote_copy(
                src_ref=partial_ref.at[pl.ds(0, CHUNK), :],
                dst_ref=rs_buf.at[s_id],
                send_sem=ssem.at[o - 1],
                recv_sem=rsem1.at[s_id],
                device_id=(me,),
                device_id_type=pl.DeviceIdType.MESH,
            )
            recv.wait_recv()

        acc = jnp.sum(rs_buf[...], axis=0)

        for r in sends1:
            r.wait_send()

        ag_buf[...] = acc
        out_ref[0, pl.ds(me * CHUNK, CHUNK), :] = acc

        sends2 = []
        for o in range(1, N_DEV):
            peer = lax.rem(me + o, N_DEV)
            r = pltpu.make_async_remote_copy(
                src_ref=ag_buf,
                dst_ref=out_ref.at[0, pl.ds(me * CHUNK, CHUNK), :],
                send_sem=ssem.at[o - 1],
                recv_sem=rsem2.at[me],
                device_id=(peer,),
                device_id_type=pl.DeviceIdType.MESH,
            )
            r.start()
            sends2.append(r)

        for o in range(1, N_DEV):
            s_id = lax.rem(me + o, N_DEV)
            recv = pltpu.make_async_remote_copy(
                src_ref=ag_buf,
                dst_ref=out_ref.at[0, pl.ds(s_id * CHUNK, CHUNK), :],
                send_sem=ssem.at[o - 1],
                recv_sem=rsem2.at[s_id],
                device_id=(me,),
                device_id_type=pl.DeviceIdType.MESH,
            )
            recv.wait_recv()

        for r in sends2:
            r.wait_send()

    return pl.pallas_call(
        body,
        out_shape=jax.ShapeDtypeStruct((1, SQ, DMODEL), jnp.float32),
        in_specs=[
            pl.BlockSpec(memory_space=pltpu.VMEM),
            pl.BlockSpec(memory_space=pltpu.VMEM),
            pl.BlockSpec(memory_space=pl.ANY),
            pl.BlockSpec(memory_space=pl.ANY),
            pl.BlockSpec(memory_space=pltpu.VMEM),
        ],
        out_specs=pl.BlockSpec(memory_space=pltpu.VMEM),
        scratch_shapes=[
            pltpu.VMEM((SKV, HEADS_PER, DH), jnp.float32),
            pltpu.VMEM((SKV, HEADS_PER, DH), jnp.float32),
            pltpu.SemaphoreType.DMA((2,)),
            pltpu.VMEM((SQ, DM), jnp.bfloat16),
            pltpu.VMEM((SQ, DMODEL), jnp.float32),
            pltpu.VMEM((N_DEV, CHUNK, DMODEL), jnp.float32),
            pltpu.VMEM((CHUNK, DMODEL), jnp.float32),
            pltpu.SemaphoreType.DMA((N_DEV,)),
            pltpu.SemaphoreType.DMA((N_DEV,)),
            pltpu.SemaphoreType.DMA((N_DEV,)),
        ],
        compiler_params=pltpu.CompilerParams(
            vmem_limit_bytes=100 * 1024 * 1024,
        ),
    )(x, Wq, K_ext, V_ext, Wo)


# device time: 90107 ns/iter; 1.1787x vs baseline; 1.1787x over previous
import jax
import jax.numpy as jnp
from jax import lax
from jax.experimental import pallas as pl
from jax.experimental.pallas import tpu as pltpu

N_DEV = 16
HEADS_PER = 8
SQ = 256
SKV = 4096
DH = 128
DM = 1024
DMODEL = 1024
CHUNK = SQ // N_DEV
SCALE = 0.08838834764831843


def kernel(x, Wq, K_ext, V_ext, Wo):
    def body(x_ref, wq_ref, k_hbm, v_hbm, wo_ref, out_ref,
             k_slab, v_slab, kv_sems, ctx_ref, partial_ref, rs_buf, ag_buf,
             ssem, rsem1, rsem2):
        me = lax.axis_index("i")

        h0 = me * HEADS_PER
        ck = pltpu.make_async_copy(
            k_hbm.at[0, :, pl.ds(h0, HEADS_PER), :], k_slab, kv_sems.at[0])
        cv = pltpu.make_async_copy(
            v_hbm.at[0, :, pl.ds(h0, HEADS_PER), :], v_slab, kv_sems.at[1])
        ck.start()
        cv.start()

        xb = x_ref[0].astype(jnp.bfloat16)
        wqb = wq_ref[...].astype(jnp.bfloat16)
        q_all = jnp.dot(xb, wqb, preferred_element_type=jnp.float32)
        q_all = q_all.astype(jnp.bfloat16).reshape(SQ, HEADS_PER, DH)

        ck.wait()
        cv.wait()

        NSEL = SKV // 4
        for h in range(HEADS_PER):
            kh = k_slab[:, h, :].astype(jnp.bfloat16)
            vh = v_slab[:, h, :].astype(jnp.bfloat16)
            k4 = kh.reshape(SKV // 256, 4, 64, DH)
            v4 = vh.reshape(SKV // 256, 4, 64, DH)
            qh = q_all[:, h, :]
            for g in range(4):
                qg = qh[g * 64:(g + 1) * 64]
                ks = k4[:, g].reshape(NSEL, DH)
                vs = v4[:, g].reshape(NSEL, DH)
                s = lax.dot_general(qg, ks, (((1,), (1,)), ((), ())),
                                    preferred_element_type=jnp.float32) * SCALE
                m = jnp.max(s, axis=-1, keepdims=True)
                w = jnp.exp(s - m)
                denom = jnp.sum(w, axis=-1, keepdims=True)
                ctx_g = jnp.dot(w.astype(jnp.bfloat16), vs,
                                preferred_element_type=jnp.float32) / denom
                ctx_ref[g * 64:(g + 1) * 64, h * DH:(h + 1) * DH] = (
                    ctx_g.astype(jnp.bfloat16))

        wob = wo_ref[...].astype(jnp.bfloat16)
        partial_ref[...] = jnp.dot(ctx_ref[...], wob,
                                   preferred_element_type=jnp.float32)

        rs_buf[pl.ds(me, 1)] = partial_ref[pl.ds(me * CHUNK, CHUNK), :].reshape(
            1, CHUNK, DMODEL)

        sends1 = []
        for o in range(1, N_DEV):
            peer = lax.rem(me + o, N_DEV)
            r = pltpu.make_async_remote_copy(
                src_ref=partial_ref.at[pl.ds(peer * CHUNK, CHUNK), :],
                dst_ref=rs_buf.at[me],
                send_sem=ssem.at[o - 1],
                recv_sem=rsem1.at[me],
                device_id=(peer,),
                device_id_type=pl.DeviceIdType.MESH,
            )
            r.start()
            sends1.append(r)

        for o in range(1, N_DEV):
            s_id = lax.rem(me + o, N_DEV)
            recv = pltpu.make_async_remote_copy(
                src_ref=partial_ref.at[pl.ds(0, CHUNK), :],
                dst_ref=rs_buf.at[s_id],
                send_sem=ssem.at[o - 1],
                recv_sem=rsem1.at[s_id],
                device_id=(me,),
                device_id_type=pl.DeviceIdType.MESH,
            )
            recv.wait_recv()

        acc = jnp.sum(rs_buf[...], axis=0)

        for r in sends1:
            r.wait_send()

        ag_buf[...] = acc
        out_ref[0, pl.ds(me * CHUNK, CHUNK), :] = acc

        sends2 = []
        for o in range(1, N_DEV):
            peer = lax.rem(me + o, N_DEV)
            r = pltpu.make_async_remote_copy(
                src_ref=ag_buf,
                dst_ref=out_ref.at[0, pl.ds(me * CHUNK, CHUNK), :],
                send_sem=ssem.at[o - 1],
                recv_sem=rsem2.at[me],
                device_id=(peer,),
                device_id_type=pl.DeviceIdType.MESH,
            )
            r.start()
            sends2.append(r)

        for o in range(1, N_DEV):
            s_id = lax.rem(me + o, N_DEV)
            recv = pltpu.make_async_remote_copy(
                src_ref=ag_buf,
                dst_ref=out_ref.at[0, pl.ds(s_id * CHUNK, CHUNK), :],
                send_sem=ssem.at[o - 1],
                recv_sem=rsem2.at[s_id],
                device_id=(me,),
                device_id_type=pl.DeviceIdType.MESH,
            )
            recv.wait_recv()

        for r in sends2:
            r.wait_send()

    return pl.pallas_call(
        body,
        out_shape=jax.ShapeDtypeStruct((1, SQ, DMODEL), jnp.float32),
        in_specs=[
            pl.BlockSpec(memory_space=pltpu.VMEM),
            pl.BlockSpec(memory_space=pltpu.VMEM),
            pl.BlockSpec(memory_space=pl.ANY),
            pl.BlockSpec(memory_space=pl.ANY),
            pl.BlockSpec(memory_space=pltpu.VMEM),
        ],
        out_specs=pl.BlockSpec(memory_space=pltpu.VMEM),
        scratch_shapes=[
            pltpu.VMEM((SKV, HEADS_PER, DH), jnp.float32),
            pltpu.VMEM((SKV, HEADS_PER, DH), jnp.float32),
            pltpu.SemaphoreType.DMA((2,)),
            pltpu.VMEM((SQ, DM), jnp.bfloat16),
            pltpu.VMEM((SQ, DMODEL), jnp.float32),
            pltpu.VMEM((N_DEV, CHUNK, DMODEL), jnp.float32),
            pltpu.VMEM((CHUNK, DMODEL), jnp.float32),
            pltpu.SemaphoreType.DMA((N_DEV,)),
            pltpu.SemaphoreType.DMA((N_DEV,)),
            pltpu.SemaphoreType.DMA((N_DEV,)),
        ],
        compiler_params=pltpu.CompilerParams(
            vmem_limit_bytes=100 * 1024 * 1024,
        ),
    )(x, Wq, K_ext, V_ext, Wo)


# device time: 55992 ns/iter; 1.8969x vs baseline; 1.6093x over previous
import jax
import jax.numpy as jnp
from jax import lax
from jax.experimental import pallas as pl
from jax.experimental.pallas import tpu as pltpu

N_DEV = 16
HEADS_PER = 8
SQ = 256
SKV = 4096
DH = 128
DM = 1024
DMODEL = 1024
CHUNK = SQ // N_DEV
SCALE = 0.08838834764831843


def kernel(x, Wq, K_ext, V_ext, Wo):
    def body(x_ref, wq_ref, k_hbm, v_hbm, wo_ref, out_ref,
             k_slab, v_slab, kv_sems, ctx_ref, partial_ref, rs_buf, ag_buf,
             ssem, rsem1, rsem2):
        me = lax.axis_index("i")

        h0 = me * HEADS_PER
        kcopies = []
        vcopies = []
        for h in range(HEADS_PER):
            ck = pltpu.make_async_copy(
                k_hbm.at[0, :, h0 + h, :], k_slab.at[h], kv_sems.at[0, h])
            cv = pltpu.make_async_copy(
                v_hbm.at[0, :, h0 + h, :], v_slab.at[h], kv_sems.at[1, h])
            ck.start()
            cv.start()
            kcopies.append(ck)
            vcopies.append(cv)

        xb = x_ref[0].astype(jnp.bfloat16)
        wqb = wq_ref[...].astype(jnp.bfloat16)
        q_all = jnp.dot(xb, wqb, preferred_element_type=jnp.float32)
        q_all = q_all.astype(jnp.bfloat16).reshape(SQ, HEADS_PER, DH)

        NSEL = SKV // 4
        for h in range(HEADS_PER):
            kcopies[h].wait()
            vcopies[h].wait()
            kh = k_slab[h].astype(jnp.bfloat16)
            vh = v_slab[h].astype(jnp.bfloat16)
            k4 = kh.reshape(SKV // 256, 4, 64, DH)
            v4 = vh.reshape(SKV // 256, 4, 64, DH)
            qh = q_all[:, h, :]
            for g in range(4):
                qg = qh[g * 64:(g + 1) * 64]
                ks = k4[:, g].reshape(NSEL, DH)
                vs = v4[:, g].reshape(NSEL, DH)
                s = lax.dot_general(qg, ks, (((1,), (1,)), ((), ())),
                                    preferred_element_type=jnp.float32) * SCALE
                m = jnp.max(s, axis=-1, keepdims=True)
                w = jnp.exp(s - m)
                denom = jnp.sum(w, axis=-1, keepdims=True)
                ctx_g = jnp.dot(w.astype(jnp.bfloat16), vs,
                                preferred_element_type=jnp.float32) / denom
                ctx_ref[g * 64:(g + 1) * 64, h * DH:(h + 1) * DH] = (
                    ctx_g.astype(jnp.bfloat16))

        wob = wo_ref[...].astype(jnp.bfloat16)
        partial_ref[...] = jnp.dot(ctx_ref[...], wob,
                                   preferred_element_type=jnp.float32)

        rs_buf[pl.ds(me, 1)] = partial_ref[pl.ds(me * CHUNK, CHUNK), :].reshape(
            1, CHUNK, DMODEL)

        sends1 = []
        for o in range(1, N_DEV):
            peer = lax.rem(me + o, N_DEV)
            r = pltpu.make_async_remote_copy(
                src_ref=partial_ref.at[pl.ds(peer * CHUNK, CHUNK), :],
                dst_ref=rs_buf.at[me],
                send_sem=ssem.at[o - 1],
                recv_sem=rsem1.at[me],
                device_id=(peer,),
                device_id_type=pl.DeviceIdType.MESH,
            )
            r.start()
            sends1.append(r)

        for o in range(1, N_DEV):
            s_id = lax.rem(me + o, N_DEV)
            recv = pltpu.make_async_remote_copy(
                src_ref=partial_ref.at[pl.ds(0, CHUNK), :],
                dst_ref=rs_buf.at[s_id],
                send_sem=ssem.at[o - 1],
                recv_sem=rsem1.at[s_id],
                device_id=(me,),
                device_id_type=pl.DeviceIdType.MESH,
            )
            recv.wait_recv()

        acc = jnp.sum(rs_buf[...], axis=0)

        for r in sends1:
            r.wait_send()

        ag_buf[...] = acc
        out_ref[0, pl.ds(me * CHUNK, CHUNK), :] = acc

        sends2 = []
        for o in range(1, N_DEV):
            peer = lax.rem(me + o, N_DEV)
            r = pltpu.make_async_remote_copy(
                src_ref=ag_buf,
                dst_ref=out_ref.at[0, pl.ds(me * CHUNK, CHUNK), :],
                send_sem=ssem.at[o - 1],
                recv_sem=rsem2.at[me],
                device_id=(peer,),
                device_id_type=pl.DeviceIdType.MESH,
            )
            r.start()
            sends2.append(r)

        for o in range(1, N_DEV):
            s_id = lax.rem(me + o, N_DEV)
            recv = pltpu.make_async_remote_copy(
                src_ref=ag_buf,
                dst_ref=out_ref.at[0, pl.ds(s_id * CHUNK, CHUNK), :],
                send_sem=ssem.at[o - 1],
                recv_sem=rsem2.at[s_id],
                device_id=(me,),
                device_id_type=pl.DeviceIdType.MESH,
            )
            recv.wait_recv()

        for r in sends2:
            r.wait_send()

    return pl.pallas_call(
        body,
        out_shape=jax.ShapeDtypeStruct((1, SQ, DMODEL), jnp.float32),
        in_specs=[
            pl.BlockSpec(memory_space=pltpu.VMEM),
            pl.BlockSpec(memory_space=pltpu.VMEM),
            pl.BlockSpec(memory_space=pl.ANY),
            pl.BlockSpec(memory_space=pl.ANY),
            pl.BlockSpec(memory_space=pltpu.VMEM),
        ],
        out_specs=pl.BlockSpec(memory_space=pltpu.VMEM),
        scratch_shapes=[
            pltpu.VMEM((HEADS_PER, SKV, DH), jnp.float32),
            pltpu.VMEM((HEADS_PER, SKV, DH), jnp.float32),
            pltpu.SemaphoreType.DMA((2, HEADS_PER)),
            pltpu.VMEM((SQ, DM), jnp.bfloat16),
            pltpu.VMEM((SQ, DMODEL), jnp.float32),
            pltpu.VMEM((N_DEV, CHUNK, DMODEL), jnp.float32),
            pltpu.VMEM((CHUNK, DMODEL), jnp.float32),
            pltpu.SemaphoreType.DMA((N_DEV,)),
            pltpu.SemaphoreType.DMA((N_DEV,)),
            pltpu.SemaphoreType.DMA((N_DEV,)),
        ],
        compiler_params=pltpu.CompilerParams(
            vmem_limit_bytes=100 * 1024 * 1024,
        ),
    )(x, Wq, K_ext, V_ext, Wo)


# device time: 47467 ns/iter; 2.2376x vs baseline; 1.1796x over previous
import os

import jax
import jax.numpy as jnp
from jax import lax
from jax.experimental import pallas as pl
from jax.experimental.pallas import tpu as pltpu

_VARIANT = os.environ.get("KERNEL_VARIANT", "full")

N_DEV = 16
HEADS_PER = 8
SQ = 256
SKV = 4096
DH = 128
DM = 1024
DMODEL = 1024
CHUNK = SQ // N_DEV
SCALE = 0.08838834764831843


def kernel(x, Wq, K_ext, V_ext, Wo):
    def body(x_ref, wq_ref, k_hbm, v_hbm, wo_ref, out_ref,
             k_slab, v_slab, kv_sems, ctx_ref, partial_ref, partial_bf,
             rs_buf, ag_buf, out_bf, ssem, rsem1, rsem2):
        me = lax.axis_index("i")

        if _VARIANT == "ar_only":
            partial_ref[...] = jnp.zeros((SQ, DMODEL), jnp.float32)
        else:
            _compute(me, x_ref, wq_ref, k_hbm, v_hbm, wo_ref,
                     k_slab, v_slab, kv_sems, ctx_ref, partial_ref)

        if _VARIANT == "no_ar":
            out_ref[0] = partial_ref[...]
            return

        partial_bf[...] = partial_ref[...].astype(jnp.bfloat16)

        sends1 = []
        for o in range(1, N_DEV):
            peer = lax.rem(me + o, N_DEV)
            r = pltpu.make_async_remote_copy(
                src_ref=partial_bf.at[pl.ds(peer * CHUNK, CHUNK), :],
                dst_ref=rs_buf.at[me],
                send_sem=ssem.at[o - 1],
                recv_sem=rsem1.at[me],
                device_id=(peer,),
                device_id_type=pl.DeviceIdType.MESH,
            )
            r.start()
            sends1.append(r)

        rs_buf[pl.ds(me, 1)] = partial_bf[pl.ds(me * CHUNK, CHUNK), :].reshape(
            1, CHUNK, DMODEL)

        for o in range(1, N_DEV):
            s_id = lax.rem(me + o, N_DEV)
            recv = pltpu.make_async_remote_copy(
                src_ref=partial_bf.at[pl.ds(0, CHUNK), :],
                dst_ref=rs_buf.at[s_id],
                send_sem=ssem.at[o - 1],
                recv_sem=rsem1.at[s_id],
                device_id=(me,),
                device_id_type=pl.DeviceIdType.MESH,
            )
            recv.wait_recv()

        acc = jnp.sum(rs_buf[...].astype(jnp.float32), axis=0)

        for r in sends1:
            r.wait_send()

        ag_buf[...] = acc.astype(jnp.bfloat16)
        out_bf[pl.ds(me * CHUNK, CHUNK), :] = ag_buf[...]

        sends2 = []
        for o in range(1, N_DEV):
            peer = lax.rem(me + o, N_DEV)
            r = pltpu.make_async_remote_copy(
                src_ref=ag_buf,
                dst_ref=out_bf.at[pl.ds(me * CHUNK, CHUNK), :],
                send_sem=ssem.at[o - 1],
                recv_sem=rsem2.at[me],
                device_id=(peer,),
                device_id_type=pl.DeviceIdType.MESH,
            )
            r.start()
            sends2.append(r)

        for o in range(1, N_DEV):
            s_id = lax.rem(me + o, N_DEV)
            recv = pltpu.make_async_remote_copy(
                src_ref=ag_buf,
                dst_ref=out_bf.at[pl.ds(s_id * CHUNK, CHUNK), :],
                send_sem=ssem.at[o - 1],
                recv_sem=rsem2.at[s_id],
                device_id=(me,),
                device_id_type=pl.DeviceIdType.MESH,
            )
            recv.wait_recv()

        out_ref[0] = out_bf[...].astype(jnp.float32)

        for r in sends2:
            r.wait_send()

    def _compute(me, x_ref, wq_ref, k_hbm, v_hbm, wo_ref,
                 k_slab, v_slab, kv_sems, ctx_ref, partial_ref):
        h0 = me * HEADS_PER
        kcopies = []
        vcopies = []
        for h in range(HEADS_PER):
            ck = pltpu.make_async_copy(
                k_hbm.at[0, :, h0 + h, :], k_slab.at[h], kv_sems.at[0, h])
            cv = pltpu.make_async_copy(
                v_hbm.at[0, :, h0 + h, :], v_slab.at[h], kv_sems.at[1, h])
            ck.start()
            cv.start()
            kcopies.append(ck)
            vcopies.append(cv)

        xb = x_ref[0].astype(jnp.bfloat16)
        wqb = wq_ref[...].astype(jnp.bfloat16)
        q_all = jnp.dot(xb, wqb, preferred_element_type=jnp.float32)
        q_all = q_all.astype(jnp.bfloat16).reshape(SQ, HEADS_PER, DH)

        NSEL = SKV // 4
        for h in range(HEADS_PER):
            kcopies[h].wait()
            vcopies[h].wait()
            kh = k_slab[h].astype(jnp.bfloat16)
            vh = v_slab[h].astype(jnp.bfloat16)
            k4 = kh.reshape(SKV // 256, 4, 64, DH)
            v4 = vh.reshape(SKV // 256, 4, 64, DH)
            qh = q_all[:, h, :]
            for g in range(4):
                qg = qh[g * 64:(g + 1) * 64]
                ks = k4[:, g].reshape(NSEL, DH)
                vs = v4[:, g].reshape(NSEL, DH)
                s = lax.dot_general(qg, ks, (((1,), (1,)), ((), ())),
                                    preferred_element_type=jnp.float32) * SCALE
                m = jnp.max(s, axis=-1, keepdims=True)
                w = jnp.exp(s - m)
                denom = jnp.sum(w, axis=-1, keepdims=True)
                ctx_g = jnp.dot(w.astype(jnp.bfloat16), vs,
                                preferred_element_type=jnp.float32) / denom
                ctx_ref[g * 64:(g + 1) * 64, h * DH:(h + 1) * DH] = (
                    ctx_g.astype(jnp.bfloat16))

        wob = wo_ref[...].astype(jnp.bfloat16)
        partial_ref[...] = jnp.dot(ctx_ref[...], wob,
                                   preferred_element_type=jnp.float32)

    return pl.pallas_call(
        body,
        out_shape=jax.ShapeDtypeStruct((1, SQ, DMODEL), jnp.float32),
        in_specs=[
            pl.BlockSpec(memory_space=pltpu.VMEM),
            pl.BlockSpec(memory_space=pltpu.VMEM),
            pl.BlockSpec(memory_space=pl.ANY),
            pl.BlockSpec(memory_space=pl.ANY),
            pl.BlockSpec(memory_space=pltpu.VMEM),
        ],
        out_specs=pl.BlockSpec(memory_space=pltpu.VMEM),
        scratch_shapes=[
            pltpu.VMEM((HEADS_PER, SKV, DH), jnp.float32),
            pltpu.VMEM((HEADS_PER, SKV, DH), jnp.float32),
            pltpu.SemaphoreType.DMA((2, HEADS_PER)),
            pltpu.VMEM((SQ, DM), jnp.bfloat16),
            pltpu.VMEM((SQ, DMODEL), jnp.float32),
            pltpu.VMEM((SQ, DMODEL), jnp.bfloat16),
            pltpu.VMEM((N_DEV, CHUNK, DMODEL), jnp.bfloat16),
            pltpu.VMEM((CHUNK, DMODEL), jnp.bfloat16),
            pltpu.VMEM((SQ, DMODEL), jnp.bfloat16),
            pltpu.SemaphoreType.DMA((N_DEV,)),
            pltpu.SemaphoreType.DMA((N_DEV,)),
            pltpu.SemaphoreType.DMA((N_DEV,)),
        ],
        compiler_params=pltpu.CompilerParams(
            vmem_limit_bytes=100 * 1024 * 1024,
        ),
    )(x, Wq, K_ext, V_ext, Wo)


# device time: 45791 ns/iter; 2.3195x vs baseline; 1.0366x over previous
import os

import jax
import jax.numpy as jnp
from jax import lax
from jax.experimental import pallas as pl
from jax.experimental.pallas import tpu as pltpu

_VARIANT = os.environ.get("KERNEL_VARIANT", "full")

N_DEV = 16
HEADS_PER = 8
SQ = 256
SKV = 4096
DH = 128
DM = 1024
DMODEL = 1024
CHUNK = SQ // N_DEV
SCALE = 0.08838834764831843


def kernel(x, Wq, K_ext, V_ext, Wo):
    def body(x_ref, wq_ref, k_hbm, v_hbm, wo_ref, out_ref,
             k_slab, v_slab, kv_sems, ctx_ref, partial_ref, partial_bf,
             bufA, colsrc, bufB, agb, colblk, out_bf, ssem, rsem1, rsem2):
        me = lax.axis_index("i")

        if _VARIANT == "ar_only":
            partial_ref[...] = jnp.zeros((SQ, DMODEL), jnp.float32)
        else:
            _compute(me, x_ref, wq_ref, k_hbm, v_hbm, wo_ref,
                     k_slab, v_slab, kv_sems, ctx_ref, partial_ref)

        if _VARIANT == "no_ar":
            out_ref[0] = partial_ref[...]
            return

        pos = lax.rem(me, 4)
        z = me // 4

        barrier_sem = pltpu.get_barrier_semaphore()
        for o in range(1, 4):
            pid = z * 4 + lax.rem(pos + o, 4)
            cid = lax.rem(z + o, 4) * 4 + pos
            pl.semaphore_signal(barrier_sem, inc=1, device_id=(pid,),
                                device_id_type=pl.DeviceIdType.MESH)
            pl.semaphore_signal(barrier_sem, inc=1, device_id=(cid,),
                                device_id_type=pl.DeviceIdType.MESH)
        pl.semaphore_wait(barrier_sem, 6)

        partial_bf[...] = partial_ref[...].astype(jnp.bfloat16)

        sends = []
        for o in range(1, 4):
            ppos = lax.rem(pos + o, 4)
            r = pltpu.make_async_remote_copy(
                src_ref=partial_bf.at[pl.ds(ppos * 64, 64), :],
                dst_ref=bufA.at[pos],
                send_sem=ssem.at[o - 1],
                recv_sem=rsem1.at[pos],
                device_id=(z * 4 + ppos,),
                device_id_type=pl.DeviceIdType.MESH,
            )
            r.start()
            sends.append(r)
        bufA[pl.ds(pos, 1)] = partial_bf[pl.ds(pos * 64, 64), :].reshape(
            1, 64, DMODEL)
        for o in range(1, 4):
            spos = lax.rem(pos + o, 4)
            pltpu.make_async_remote_copy(
                src_ref=partial_bf.at[pl.ds(0, 64), :],
                dst_ref=bufA.at[spos],
                send_sem=ssem.at[o - 1],
                recv_sem=rsem1.at[spos],
                device_id=(me,),
                device_id_type=pl.DeviceIdType.MESH,
            ).wait_recv()
        accA = jnp.sum(bufA[...].astype(jnp.float32), axis=0)
        for r in sends:
            r.wait_send()

        colsrc[...] = accA.astype(jnp.bfloat16)
        sends = []
        for o in range(1, 4):
            zz = lax.rem(z + o, 4)
            r = pltpu.make_async_remote_copy(
                src_ref=colsrc.at[pl.ds(zz * CHUNK, CHUNK), :],
                dst_ref=bufB.at[z],
                send_sem=ssem.at[3 + o - 1],
                recv_sem=rsem1.at[4 + z],
                device_id=(zz * 4 + pos,),
                device_id_type=pl.DeviceIdType.MESH,
            )
            r.start()
            sends.append(r)
        bufB[pl.ds(z, 1)] = colsrc[pl.ds(z * CHUNK, CHUNK), :].reshape(
            1, CHUNK, DMODEL)
        for o in range(1, 4):
            sz = lax.rem(z + o, 4)
            pltpu.make_async_remote_copy(
                src_ref=colsrc.at[pl.ds(0, CHUNK), :],
                dst_ref=bufB.at[sz],
                send_sem=ssem.at[3 + o - 1],
                recv_sem=rsem1.at[4 + sz],
                device_id=(me,),
                device_id_type=pl.DeviceIdType.MESH,
            ).wait_recv()
        accB = jnp.sum(bufB[...].astype(jnp.float32), axis=0)
        for r in sends:
            r.wait_send()

        agb[...] = accB.astype(jnp.bfloat16)
        sends = []
        for o in range(1, 4):
            zz = lax.rem(z + o, 4)
            r = pltpu.make_async_remote_copy(
                src_ref=agb,
                dst_ref=colblk.at[pl.ds(z * CHUNK, CHUNK), :],
                send_sem=ssem.at[6 + o - 1],
                recv_sem=rsem2.at[z],
                device_id=(zz * 4 + pos,),
                device_id_type=pl.DeviceIdType.MESH,
            )
            r.start()
            sends.append(r)
        colblk[pl.ds(z * CHUNK, CHUNK), :] = agb[...]
        for o in range(1, 4):
            sz = lax.rem(z + o, 4)
            pltpu.make_async_remote_copy(
                src_ref=agb,
                dst_ref=colblk.at[pl.ds(sz * CHUNK, CHUNK), :],
                send_sem=ssem.at[6 + o - 1],
                recv_sem=rsem2.at[sz],
                device_id=(me,),
                device_id_type=pl.DeviceIdType.MESH,
            ).wait_recv()
        for r in sends:
            r.wait_send()

        sends = []
        for o in range(1, 4):
            ppos = lax.rem(pos + o, 4)
            r = pltpu.make_async_remote_copy(
                src_ref=colblk,
                dst_ref=out_bf.at[pl.ds(pos * 64, 64), :],
                send_sem=ssem.at[9 + o - 1],
                recv_sem=rsem2.at[4 + pos],
                device_id=(z * 4 + ppos,),
                device_id_type=pl.DeviceIdType.MESH,
            )
            r.start()
            sends.append(r)
        out_bf[pl.ds(pos * 64, 64), :] = colblk[...]
        for o in range(1, 4):
            spos = lax.rem(pos + o, 4)
            pltpu.make_async_remote_copy(
                src_ref=colblk,
                dst_ref=out_bf.at[pl.ds(spos * 64, 64), :],
                send_sem=ssem.at[9 + o - 1],
                recv_sem=rsem2.at[4 + spos],
                device_id=(me,),
                device_id_type=pl.DeviceIdType.MESH,
            ).wait_recv()

        out_ref[0] = out_bf[...].astype(jnp.float32)

        for r in sends:
            r.wait_send()

    def _compute(me, x_ref, wq_ref, k_hbm, v_hbm, wo_ref,
                 k_slab, v_slab, kv_sems, ctx_ref, partial_ref):
        h0 = me * HEADS_PER
        kcopies = []
        vcopies = []
        for h in range(HEADS_PER):
            ck = pltpu.make_async_copy(
                k_hbm.at[0, :, h0 + h, :], k_slab.at[h], kv_sems.at[0, h])
            cv = pltpu.make_async_copy(
                v_hbm.at[0, :, h0 + h, :], v_slab.at[h], kv_sems.at[1, h])
            ck.start()
            cv.start()
            kcopies.append(ck)
            vcopies.append(cv)

        xb = x_ref[0].astype(jnp.bfloat16)
        wqb = wq_ref[...].astype(jnp.bfloat16)
        q_all = jnp.dot(xb, wqb, preferred_element_type=jnp.float32)
        q_all = q_all.astype(jnp.bfloat16).reshape(SQ, HEADS_PER, DH)

        NSEL = SKV // 4
        for h in range(HEADS_PER):
            kcopies[h].wait()
            vcopies[h].wait()
            kh = k_slab[h].astype(jnp.bfloat16)
            vh = v_slab[h].astype(jnp.bfloat16)
            k4 = kh.reshape(SKV // 256, 4, 64, DH)
            v4 = vh.reshape(SKV // 256, 4, 64, DH)
            qh = q_all[:, h, :]
            for g in range(4):
                qg = qh[g * 64:(g + 1) * 64]
                ks = k4[:, g].reshape(NSEL, DH)
                vs = v4[:, g].reshape(NSEL, DH)
                s = lax.dot_general(qg, ks, (((1,), (1,)), ((), ())),
                                    preferred_element_type=jnp.float32) * SCALE
                m = jnp.max(s, axis=-1, keepdims=True)
                w = jnp.exp(s - m)
                denom = jnp.sum(w, axis=-1, keepdims=True)
                ctx_g = jnp.dot(w.astype(jnp.bfloat16), vs,
                                preferred_element_type=jnp.float32) / denom
                ctx_ref[g * 64:(g + 1) * 64, h * DH:(h + 1) * DH] = (
                    ctx_g.astype(jnp.bfloat16))

        wob = wo_ref[...].astype(jnp.bfloat16)
        partial_ref[...] = jnp.dot(ctx_ref[...], wob,
                                   preferred_element_type=jnp.float32)

    return pl.pallas_call(
        body,
        out_shape=jax.ShapeDtypeStruct((1, SQ, DMODEL), jnp.float32),
        in_specs=[
            pl.BlockSpec(memory_space=pltpu.VMEM),
            pl.BlockSpec(memory_space=pltpu.VMEM),
            pl.BlockSpec(memory_space=pl.ANY),
            pl.BlockSpec(memory_space=pl.ANY),
            pl.BlockSpec(memory_space=pltpu.VMEM),
        ],
        out_specs=pl.BlockSpec(memory_space=pltpu.VMEM),
        scratch_shapes=[
            pltpu.VMEM((HEADS_PER, SKV, DH), jnp.float32),
            pltpu.VMEM((HEADS_PER, SKV, DH), jnp.float32),
            pltpu.SemaphoreType.DMA((2, HEADS_PER)),
            pltpu.VMEM((SQ, DM), jnp.bfloat16),
            pltpu.VMEM((SQ, DMODEL), jnp.float32),
            pltpu.VMEM((SQ, DMODEL), jnp.bfloat16),
            pltpu.VMEM((4, 64, DMODEL), jnp.bfloat16),
            pltpu.VMEM((64, DMODEL), jnp.bfloat16),
            pltpu.VMEM((4, CHUNK, DMODEL), jnp.bfloat16),
            pltpu.VMEM((CHUNK, DMODEL), jnp.bfloat16),
            pltpu.VMEM((64, DMODEL), jnp.bfloat16),
            pltpu.VMEM((SQ, DMODEL), jnp.bfloat16),
            pltpu.SemaphoreType.DMA((12,)),
            pltpu.SemaphoreType.DMA((8,)),
            pltpu.SemaphoreType.DMA((8,)),
        ],
        compiler_params=pltpu.CompilerParams(
            vmem_limit_bytes=100 * 1024 * 1024,
            collective_id=0,
        ),
    )(x, Wq, K_ext, V_ext, Wo)


# device time: 43858 ns/iter; 2.4217x vs baseline; 1.0441x over previous
import os

import jax
import jax.numpy as jnp
from jax import lax
from jax.experimental import pallas as pl
from jax.experimental.pallas import tpu as pltpu

_VARIANT = os.environ.get("KERNEL_VARIANT", "full")

N_DEV = 16
HEADS_PER = 8
SQ = 256
SKV = 4096
DH = 128
DM = 1024
DMODEL = 1024
CHUNK = SQ // N_DEV
SCALE = 0.08838834764831843


def kernel(x, Wq, K_ext, V_ext, Wo):
    def body(x_ref, wq_ref, k_hbm, v_hbm, wo_ref, out_ref,
             k_slab, v_slab, kv_sems, ctx_ref, partial_ref, partial_bf,
             bufA, colsrc, bufB, agb, colblk, out_bf, ssem, rsem1, rsem2):
        me = lax.axis_index("i")
        pos = lax.rem(me, 4)
        z = me // 4
        do_ar = _VARIANT != "no_ar"

        if do_ar:
            barrier_sem = pltpu.get_barrier_semaphore()
            for o in range(1, 4):
                pid = z * 4 + lax.rem(pos + o, 4)
                cid = lax.rem(z + o, 4) * 4 + pos
                pl.semaphore_signal(barrier_sem, inc=1, device_id=(pid,),
                                    device_id_type=pl.DeviceIdType.MESH)
                pl.semaphore_signal(barrier_sem, inc=1, device_id=(cid,),
                                    device_id_type=pl.DeviceIdType.MESH)

        if _VARIANT == "ar_only":
            ctx_ref[...] = jnp.zeros((SQ, DM), jnp.bfloat16)
        else:
            _attention(me, x_ref, wq_ref, k_hbm, v_hbm,
                       k_slab, v_slab, kv_sems, ctx_ref)

        wob = wo_ref[...].astype(jnp.bfloat16)
        if do_ar:
            pl.semaphore_wait(barrier_sem, 6)
        sendsA = []
        for o in range(1, 4):
            ppos = lax.rem(pos + o, 4)
            blk = jnp.dot(ctx_ref[pl.ds(ppos * 64, 64), :], wob,
                          preferred_element_type=jnp.float32)
            partial_ref[pl.ds(ppos * 64, 64), :] = blk
            if do_ar:
                partial_bf[pl.ds(ppos * 64, 64), :] = blk.astype(jnp.bfloat16)
                r = pltpu.make_async_remote_copy(
                    src_ref=partial_bf.at[pl.ds(ppos * 64, 64), :],
                    dst_ref=bufA.at[pos],
                    send_sem=ssem.at[o - 1],
                    recv_sem=rsem1.at[pos],
                    device_id=(z * 4 + ppos,),
                    device_id_type=pl.DeviceIdType.MESH,
                )
                r.start()
                sendsA.append(r)
        own_blk = jnp.dot(ctx_ref[pl.ds(pos * 64, 64), :], wob,
                          preferred_element_type=jnp.float32)
        partial_ref[pl.ds(pos * 64, 64), :] = own_blk

        if not do_ar:
            out_ref[0] = partial_ref[...]
            return
        bufA[pl.ds(pos, 1)] = own_blk.astype(jnp.bfloat16).reshape(
            1, 64, DMODEL)

        for o in range(1, 4):
            spos = lax.rem(pos + o, 4)
            pltpu.make_async_remote_copy(
                src_ref=partial_bf.at[pl.ds(0, 64), :],
                dst_ref=bufA.at[spos],
                send_sem=ssem.at[o - 1],
                recv_sem=rsem1.at[spos],
                device_id=(me,),
                device_id_type=pl.DeviceIdType.MESH,
            ).wait_recv()
        accA = jnp.sum(bufA[...].astype(jnp.float32), axis=0)
        for r in sendsA:
            r.wait_send()

        colsrc[...] = accA.astype(jnp.bfloat16)
        sends = []
        for o in range(1, 4):
            zz = lax.rem(z + o, 4)
            r = pltpu.make_async_remote_copy(
                src_ref=colsrc.at[pl.ds(zz * CHUNK, CHUNK), :],
                dst_ref=bufB.at[z],
                send_sem=ssem.at[3 + o - 1],
                recv_sem=rsem1.at[4 + z],
                device_id=(zz * 4 + pos,),
                device_id_type=pl.DeviceIdType.MESH,
            )
            r.start()
            sends.append(r)
        bufB[pl.ds(z, 1)] = colsrc[pl.ds(z * CHUNK, CHUNK), :].reshape(
            1, CHUNK, DMODEL)
        for o in range(1, 4):
            sz = lax.rem(z + o, 4)
            pltpu.make_async_remote_copy(
                src_ref=colsrc.at[pl.ds(0, CHUNK), :],
                dst_ref=bufB.at[sz],
                send_sem=ssem.at[3 + o - 1],
                recv_sem=rsem1.at[4 + sz],
                device_id=(me,),
                device_id_type=pl.DeviceIdType.MESH,
            ).wait_recv()
        accB = jnp.sum(bufB[...].astype(jnp.float32), axis=0)
        for r in sends:
            r.wait_send()

        agb[...] = accB.astype(jnp.bfloat16)
        sends = []
        for o in range(1, 4):
            zz = lax.rem(z + o, 4)
            r = pltpu.make_async_remote_copy(
                src_ref=agb,
                dst_ref=colblk.at[pl.ds(z * CHUNK, CHUNK), :],
                send_sem=ssem.at[6 + o - 1],
                recv_sem=rsem2.at[z],
                device_id=(zz * 4 + pos,),
                device_id_type=pl.DeviceIdType.MESH,
            )
            r.start()
            sends.append(r)
        colblk[pl.ds(z * CHUNK, CHUNK), :] = agb[...]
        for o in range(1, 4):
            sz = lax.rem(z + o, 4)
            pltpu.make_async_remote_copy(
                src_ref=agb,
                dst_ref=colblk.at[pl.ds(sz * CHUNK, CHUNK), :],
                send_sem=ssem.at[6 + o - 1],
                recv_sem=rsem2.at[sz],
                device_id=(me,),
                device_id_type=pl.DeviceIdType.MESH,
            ).wait_recv()
        for r in sends:
            r.wait_send()

        sends = []
        for o in range(1, 4):
            ppos = lax.rem(pos + o, 4)
            r = pltpu.make_async_remote_copy(
                src_ref=colblk,
                dst_ref=out_bf.at[pl.ds(pos * 64, 64), :],
                send_sem=ssem.at[9 + o - 1],
                recv_sem=rsem2.at[4 + pos],
                device_id=(z * 4 + ppos,),
                device_id_type=pl.DeviceIdType.MESH,
            )
            r.start()
            sends.append(r)
        out_bf[pl.ds(pos * 64, 64), :] = colblk[...]
        for o in range(1, 4):
            spos = lax.rem(pos + o, 4)
            pltpu.make_async_remote_copy(
                src_ref=colblk,
                dst_ref=out_bf.at[pl.ds(spos * 64, 64), :],
                send_sem=ssem.at[9 + o - 1],
                recv_sem=rsem2.at[4 + spos],
                device_id=(me,),
                device_id_type=pl.DeviceIdType.MESH,
            ).wait_recv()

        out_ref[0] = out_bf[...].astype(jnp.float32)

        for r in sends:
            r.wait_send()

    def _attention(me, x_ref, wq_ref, k_hbm, v_hbm,
                   k_slab, v_slab, kv_sems, ctx_ref):
        h0 = me * HEADS_PER
        kcopies = []
        vcopies = []
        for h in range(HEADS_PER):
            ck = pltpu.make_async_copy(
                k_hbm.at[0, :, h0 + h, :], k_slab.at[h], kv_sems.at[0, h])
            cv = pltpu.make_async_copy(
                v_hbm.at[0, :, h0 + h, :], v_slab.at[h], kv_sems.at[1, h])
            ck.start()
            cv.start()
            kcopies.append(ck)
            vcopies.append(cv)

        xb = x_ref[0].astype(jnp.bfloat16)
        wqb = wq_ref[...].astype(jnp.bfloat16)
        q_all = jnp.dot(xb, wqb, preferred_element_type=jnp.float32)
        q_all = q_all.astype(jnp.bfloat16).reshape(SQ, HEADS_PER, DH)

        NSEL = SKV // 4
        for h in range(HEADS_PER):
            kcopies[h].wait()
            vcopies[h].wait()
            kh = k_slab[h].astype(jnp.bfloat16)
            vh = v_slab[h].astype(jnp.bfloat16)
            k4 = kh.reshape(SKV // 256, 4, 64, DH)
            v4 = vh.reshape(SKV // 256, 4, 64, DH)
            qh = q_all[:, h, :]
            for g in range(4):
                qg = qh[g * 64:(g + 1) * 64]
                ks = k4[:, g].reshape(NSEL, DH)
                vs = v4[:, g].reshape(NSEL, DH)
                s = lax.dot_general(qg, ks, (((1,), (1,)), ((), ())),
                                    preferred_element_type=jnp.float32) * SCALE
                m = jnp.max(s, axis=-1, keepdims=True)
                w = jnp.exp(s - m)
                denom = jnp.sum(w, axis=-1, keepdims=True)
                ctx_g = jnp.dot(w.astype(jnp.bfloat16), vs,
                                preferred_element_type=jnp.float32) / denom
                ctx_ref[g * 64:(g + 1) * 64, h * DH:(h + 1) * DH] = (
                    ctx_g.astype(jnp.bfloat16))

    return pl.pallas_call(
        body,
        out_shape=jax.ShapeDtypeStruct((1, SQ, DMODEL), jnp.float32),
        in_specs=[
            pl.BlockSpec(memory_space=pltpu.VMEM),
            pl.BlockSpec(memory_space=pltpu.VMEM),
            pl.BlockSpec(memory_space=pl.ANY),
            pl.BlockSpec(memory_space=pl.ANY),
            pl.BlockSpec(memory_space=pltpu.VMEM),
        ],
        out_specs=pl.BlockSpec(memory_space=pltpu.VMEM),
        scratch_shapes=[
            pltpu.VMEM((HEADS_PER, SKV, DH), jnp.float32),
            pltpu.VMEM((HEADS_PER, SKV, DH), jnp.float32),
            pltpu.SemaphoreType.DMA((2, HEADS_PER)),
            pltpu.VMEM((SQ, DM), jnp.bfloat16),
            pltpu.VMEM((SQ, DMODEL), jnp.float32),
            pltpu.VMEM((SQ, DMODEL), jnp.bfloat16),
            pltpu.VMEM((4, 64, DMODEL), jnp.bfloat16),
            pltpu.VMEM((64, DMODEL), jnp.bfloat16),
            pltpu.VMEM((4, CHUNK, DMODEL), jnp.bfloat16),
            pltpu.VMEM((CHUNK, DMODEL), jnp.bfloat16),
            pltpu.VMEM((64, DMODEL), jnp.bfloat16),
            pltpu.VMEM((SQ, DMODEL), jnp.bfloat16),
            pltpu.SemaphoreType.DMA((12,)),
            pltpu.SemaphoreType.DMA((8,)),
            pltpu.SemaphoreType.DMA((8,)),
        ],
        compiler_params=pltpu.CompilerParams(
            vmem_limit_bytes=100 * 1024 * 1024,
            collective_id=0,
        ),
    )(x, Wq, K_ext, V_ext, Wo)


# device time: 43819 ns/iter; 2.4239x vs baseline; 1.0009x over previous
import os

import jax
import jax.numpy as jnp
from jax import lax
from jax.experimental import pallas as pl
from jax.experimental.pallas import tpu as pltpu

_VARIANT = os.environ.get("KERNEL_VARIANT", "full")

N_DEV = 16
HEADS_PER = 8
SQ = 256
SKV = 4096
DH = 128
DM = 1024
DMODEL = 1024
CHUNK = SQ // N_DEV
SCALE = 0.08838834764831843


def kernel(x, Wq, K_ext, V_ext, Wo):
    def body(x_ref, wq_ref, k_hbm, v_hbm, wo_ref, out_ref,
             k_slab, v_slab, kv_sems, ctx_ref, partial_ref, partial_bf,
             bufA, colsrc, bufB, agb, colblk, out_bf, ssem, rsem1, rsem2):
        me = lax.axis_index("i")
        pos = lax.rem(me, 4)
        z = me // 4
        do_ar = _VARIANT not in ("no_ar", "dma_only")

        if do_ar:
            barrier_sem = pltpu.get_barrier_semaphore()
            for o in range(1, 4):
                pid = z * 4 + lax.rem(pos + o, 4)
                cid = lax.rem(z + o, 4) * 4 + pos
                pl.semaphore_signal(barrier_sem, inc=1, device_id=(pid,),
                                    device_id_type=pl.DeviceIdType.MESH)
                pl.semaphore_signal(barrier_sem, inc=1, device_id=(cid,),
                                    device_id_type=pl.DeviceIdType.MESH)

        if _VARIANT == "ar_only":
            ctx_ref[...] = jnp.zeros((SQ, DM), jnp.bfloat16)
        else:
            _attention(me, x_ref, wq_ref, k_hbm, v_hbm,
                       k_slab, v_slab, kv_sems, ctx_ref)

        wob = wo_ref[...].astype(jnp.bfloat16)
        if do_ar:
            pl.semaphore_wait(barrier_sem, 6)
        sendsA = []
        for o in range(1, 4):
            ppos = lax.rem(pos + o, 4)
            blk = jnp.dot(ctx_ref[pl.ds(ppos * 64, 64), :], wob,
                          preferred_element_type=jnp.float32)
            partial_ref[pl.ds(ppos * 64, 64), :] = blk
            if do_ar:
                partial_bf[pl.ds(ppos * 64, 64), :] = blk.astype(jnp.bfloat16)
                r = pltpu.make_async_remote_copy(
                    src_ref=partial_bf.at[pl.ds(ppos * 64, 64), :],
                    dst_ref=bufA.at[pos],
                    send_sem=ssem.at[o - 1],
                    recv_sem=rsem1.at[pos],
                    device_id=(z * 4 + ppos,),
                    device_id_type=pl.DeviceIdType.MESH,
                )
                r.start()
                sendsA.append(r)
        own_blk = jnp.dot(ctx_ref[pl.ds(pos * 64, 64), :], wob,
                          preferred_element_type=jnp.float32)
        partial_ref[pl.ds(pos * 64, 64), :] = own_blk

        if not do_ar:
            out_ref[0] = partial_ref[...]
            return
        bufA[pl.ds(pos, 1)] = own_blk.astype(jnp.bfloat16).reshape(
            1, 64, DMODEL)

        for o in range(1, 4):
            spos = lax.rem(pos + o, 4)
            pltpu.make_async_remote_copy(
                src_ref=partial_bf.at[pl.ds(0, 64), :],
                dst_ref=bufA.at[spos],
                send_sem=ssem.at[o - 1],
                recv_sem=rsem1.at[spos],
                device_id=(me,),
                device_id_type=pl.DeviceIdType.MESH,
            ).wait_recv()
        accA = jnp.sum(bufA[...].astype(jnp.float32), axis=0)
        for r in sendsA:
            r.wait_send()

        colsrc[...] = accA.astype(jnp.bfloat16)
        sends = []
        for o in range(1, 4):
            zz = lax.rem(z + o, 4)
            r = pltpu.make_async_remote_copy(
                src_ref=colsrc.at[pl.ds(zz * CHUNK, CHUNK), :],
                dst_ref=bufB.at[z],
                send_sem=ssem.at[3 + o - 1],
                recv_sem=rsem1.at[4 + z],
                device_id=(zz * 4 + pos,),
                device_id_type=pl.DeviceIdType.MESH,
            )
            r.start()
            sends.append(r)
        bufB[pl.ds(z, 1)] = colsrc[pl.ds(z * CHUNK, CHUNK), :].reshape(
            1, CHUNK, DMODEL)
        for o in range(1, 4):
            sz = lax.rem(z + o, 4)
            pltpu.make_async_remote_copy(
                src_ref=colsrc.at[pl.ds(0, CHUNK), :],
                dst_ref=bufB.at[sz],
                send_sem=ssem.at[3 + o - 1],
                recv_sem=rsem1.at[4 + sz],
                device_id=(me,),
                device_id_type=pl.DeviceIdType.MESH,
            ).wait_recv()
        accB = jnp.sum(bufB[...].astype(jnp.float32), axis=0)
        for r in sends:
            r.wait_send()

        agb[...] = accB.astype(jnp.bfloat16)
        sends = []
        for o in range(1, 4):
            zz = lax.rem(z + o, 4)
            r = pltpu.make_async_remote_copy(
                src_ref=agb,
                dst_ref=colblk.at[pl.ds(z * CHUNK, CHUNK), :],
                send_sem=ssem.at[6 + o - 1],
                recv_sem=rsem2.at[z],
                device_id=(zz * 4 + pos,),
                device_id_type=pl.DeviceIdType.MESH,
            )
            r.start()
            sends.append(r)
        colblk[pl.ds(z * CHUNK, CHUNK), :] = agb[...]
        for o in range(1, 4):
            sz = lax.rem(z + o, 4)
            pltpu.make_async_remote_copy(
                src_ref=agb,
                dst_ref=colblk.at[pl.ds(sz * CHUNK, CHUNK), :],
                send_sem=ssem.at[6 + o - 1],
                recv_sem=rsem2.at[sz],
                device_id=(me,),
                device_id_type=pl.DeviceIdType.MESH,
            ).wait_recv()
        for r in sends:
            r.wait_send()

        sends = []
        for o in range(1, 4):
            ppos = lax.rem(pos + o, 4)
            r = pltpu.make_async_remote_copy(
                src_ref=colblk,
                dst_ref=out_bf.at[pl.ds(pos * 64, 64), :],
                send_sem=ssem.at[9 + o - 1],
                recv_sem=rsem2.at[4 + pos],
                device_id=(z * 4 + ppos,),
                device_id_type=pl.DeviceIdType.MESH,
            )
            r.start()
            sends.append(r)
        out_bf[pl.ds(pos * 64, 64), :] = colblk[...]
        for o in range(1, 4):
            spos = lax.rem(pos + o, 4)
            pltpu.make_async_remote_copy(
                src_ref=colblk,
                dst_ref=out_bf.at[pl.ds(spos * 64, 64), :],
                send_sem=ssem.at[9 + o - 1],
                recv_sem=rsem2.at[4 + spos],
                device_id=(me,),
                device_id_type=pl.DeviceIdType.MESH,
            ).wait_recv()

        out_ref[0] = out_bf[...].astype(jnp.float32)

        for r in sends:
            r.wait_send()

    def _attention(me, x_ref, wq_ref, k_hbm, v_hbm,
                   k_slab, v_slab, kv_sems, ctx_ref):
        h0 = me * HEADS_PER
        kcopies = []
        vcopies = []
        for h in range(HEADS_PER):
            ck = pltpu.make_async_copy(
                k_hbm.at[0, :, h0 + h, :], k_slab.at[h], kv_sems.at[0, h])
            cv = pltpu.make_async_copy(
                v_hbm.at[0, :, h0 + h, :], v_slab.at[h], kv_sems.at[1, h])
            ck.start()
            cv.start()
            kcopies.append(ck)
            vcopies.append(cv)

        xb = x_ref[0].astype(jnp.bfloat16)
        wqb = wq_ref[...].astype(jnp.bfloat16)
        q_all = jnp.dot(xb, wqb, preferred_element_type=jnp.float32)
        q_all = q_all.astype(jnp.bfloat16).reshape(SQ, HEADS_PER, DH)

        NSEL = SKV // 4
        for h in range(HEADS_PER):
            kcopies[h].wait()
            vcopies[h].wait()
            if _VARIANT == "dma_only":
                continue
            kh = k_slab[h].astype(jnp.bfloat16)
            vh = v_slab[h].astype(jnp.bfloat16)
            k4 = kh.reshape(SKV // 256, 4, 64, DH)
            v4 = vh.reshape(SKV // 256, 4, 64, DH)
            qh = q_all[:, h, :]
            for g in range(4):
                qg = qh[g * 64:(g + 1) * 64]
                ks = k4[:, g].reshape(NSEL, DH)
                vs = v4[:, g].reshape(NSEL, DH)
                s = lax.dot_general(qg, ks, (((1,), (1,)), ((), ())),
                                    preferred_element_type=jnp.float32) * SCALE
                m = jnp.max(s, axis=-1, keepdims=True)
                w = jnp.exp(s - m)
                denom = jnp.sum(w, axis=-1, keepdims=True)
                ctx_g = jnp.dot(w.astype(jnp.bfloat16), vs,
                                preferred_element_type=jnp.float32) / denom
                ctx_ref[g * 64:(g + 1) * 64, h * DH:(h + 1) * DH] = (
                    ctx_g.astype(jnp.bfloat16))

    return pl.pallas_call(
        body,
        out_shape=jax.ShapeDtypeStruct((1, SQ, DMODEL), jnp.float32),
        in_specs=[
            pl.BlockSpec(memory_space=pltpu.VMEM),
            pl.BlockSpec(memory_space=pltpu.VMEM),
            pl.BlockSpec(memory_space=pl.ANY),
            pl.BlockSpec(memory_space=pl.ANY),
            pl.BlockSpec(memory_space=pltpu.VMEM),
        ],
        out_specs=pl.BlockSpec(memory_space=pltpu.VMEM),
        scratch_shapes=[
            pltpu.VMEM((HEADS_PER, SKV, DH), jnp.float32),
            pltpu.VMEM((HEADS_PER, SKV, DH), jnp.float32),
            pltpu.SemaphoreType.DMA((2, HEADS_PER)),
            pltpu.VMEM((SQ, DM), jnp.bfloat16),
            pltpu.VMEM((SQ, DMODEL), jnp.float32),
            pltpu.VMEM((SQ, DMODEL), jnp.bfloat16),
            pltpu.VMEM((4, 64, DMODEL), jnp.bfloat16),
            pltpu.VMEM((64, DMODEL), jnp.bfloat16),
            pltpu.VMEM((4, CHUNK, DMODEL), jnp.bfloat16),
            pltpu.VMEM((CHUNK, DMODEL), jnp.bfloat16),
            pltpu.VMEM((64, DMODEL), jnp.bfloat16),
            pltpu.VMEM((SQ, DMODEL), jnp.bfloat16),
            pltpu.SemaphoreType.DMA((12,)),
            pltpu.SemaphoreType.DMA((8,)),
            pltpu.SemaphoreType.DMA((8,)),
        ],
        compiler_params=pltpu.CompilerParams(
            vmem_limit_bytes=100 * 1024 * 1024,
            **({} if _VARIANT in ("no_ar", "dma_only") else
               {"collective_id": 0}),
        ),
    )(x, Wq, K_ext, V_ext, Wo)
